# baseline (speedup 1.0000x reference)
import functools

import jax
import jax.numpy as jnp
from jax import lax
from jax.experimental import pallas as pl
from jax.experimental.pallas import tpu as pltpu
from jax.experimental.pallas import tpu_sc as plsc

_N = 320000
_D = 128
_NC = 2
_NS = 16
_NW = _NC * _NS
_ROWS = _N // _NW
_CH = 400
_NCH = _ROWS // _CH

_mesh = plsc.VectorSubcoreMesh(core_axis_name="c", subcore_axis_name="s")


@functools.partial(
    pl.kernel,
    mesh=_mesh,
    out_type=jax.ShapeDtypeStruct((_N, _D), jnp.float32),
    scratch_types=[pltpu.VMEM((_CH, _D), jnp.float32)],
)
def _sc_copy(src_hbm, out_hbm, buf):
    wid = lax.axis_index("s") * _NC + lax.axis_index("c")
    base = wid * _ROWS

    def body(j, carry):
        off = base + j * _CH
        pltpu.sync_copy(src_hbm.at[pl.ds(off, _CH)], buf)
        pltpu.sync_copy(buf, out_hbm.at[pl.ds(off, _CH)])
        return carry

    lax.fori_loop(0, _NCH, body, 0)


def kernel(ref_bxyz, ref_feat, group_ids):
    del ref_bxyz, group_ids
    return _sc_copy(ref_feat)


# speedup vs baseline: 1.0985x; 1.0985x over previous
import functools

import jax
import jax.numpy as jnp
from jax import lax
from jax.experimental import pallas as pl
from jax.experimental.pallas import tpu as pltpu
from jax.experimental.pallas import tpu_sc as plsc

_N = 320000
_D = 128
_NC = 2
_NS = 16
_NW = _NC * _NS
_ROWS = _N // _NW
_CH = 400
_NCH = _ROWS // _CH

_mesh = plsc.VectorSubcoreMesh(core_axis_name="c", subcore_axis_name="s")


@functools.partial(
    pl.kernel,
    mesh=_mesh,
    out_type=jax.ShapeDtypeStruct((_N, _D), jnp.float32),
    scratch_types=[
        pltpu.VMEM((_CH, _D), jnp.float32),
        pltpu.VMEM((_CH, _D), jnp.float32),
        pltpu.SemaphoreType.DMA,
        pltpu.SemaphoreType.DMA,
        pltpu.SemaphoreType.DMA,
        pltpu.SemaphoreType.DMA,
    ],
)
def _sc_copy(src_hbm, out_hbm, buf0, buf1, rs0, rs1, ws0, ws1):
    bufs = (buf0, buf1)
    rsems = (rs0, rs1)
    wsems = (ws0, ws1)
    wid = lax.axis_index("s") * _NC + lax.axis_index("c")
    base = wid * _ROWS

    def rd(j):
        off = base + j * _CH
        return pltpu.async_copy(src_hbm.at[pl.ds(off, _CH)], bufs[j % 2], rsems[j % 2])

    def wr(j):
        off = base + j * _CH
        return pltpu.async_copy(bufs[j % 2], out_hbm.at[pl.ds(off, _CH)], wsems[j % 2])

    rhs = [rd(0), None]
    whs = [None, None]
    for j in range(_NCH):
        b = j % 2
        nb = (j + 1) % 2
        if j + 1 < _NCH:
            if whs[nb] is not None:
                whs[nb].wait()
            rhs[nb] = rd(j + 1)
        rhs[b].wait()
        whs[b] = wr(j)
    for h in whs:
        if h is not None:
            h.wait()


def kernel(ref_bxyz, ref_feat, group_ids):
    del ref_bxyz, group_ids
    return _sc_copy(ref_feat)
